# baseline (speedup 1.0000x reference)
import functools

import jax
import jax.numpy as jnp
from jax import lax
from jax.experimental import pallas as pl
from jax.experimental.pallas import tpu as pltpu
from jax.experimental.pallas import tpu_sc as plsc

BATCH = 16384
TOPIC_DIM = 128
SUBTOPIC_DIM = 64
OUT_DIM = TOPIC_DIM + SUBTOPIC_DIM
CHUNK = 128


@functools.cache
def _build():
    info = plsc.get_sparse_core_info()
    num_workers = info.num_cores * info.num_subcores
    rows_per_worker = BATCH // num_workers
    n_chunks = rows_per_worker // CHUNK
    mesh = plsc.VectorSubcoreMesh(core_axis_name="c", subcore_axis_name="s")

    @functools.partial(
        pl.kernel,
        mesh=mesh,
        out_type=jax.ShapeDtypeStruct((BATCH, OUT_DIM), jnp.float32),
        scratch_types=[
            pltpu.VMEM((CHUNK,), jnp.int32),
            pltpu.VMEM((CHUNK,), jnp.int32),
            pltpu.VMEM((CHUNK, TOPIC_DIM), jnp.float32),
            pltpu.VMEM((CHUNK, SUBTOPIC_DIM), jnp.float32),
            pltpu.SemaphoreType.DMA,
            pltpu.SemaphoreType.DMA,
        ],
        compiler_params=pltpu.CompilerParams(use_tc_tiling_on_sc=False),
    )
    def enc(t_idx_hbm, s_idx_hbm, t_tab_hbm, s_tab_hbm, out_hbm,
            t_idx_v, s_idx_v, t_rows_v, s_rows_v, t_sem, s_sem):
        wid = lax.axis_index("s") * info.num_cores + lax.axis_index("c")
        base = wid * rows_per_worker

        def body(c, carry):
            row0 = base + c * CHUNK
            pltpu.sync_copy(t_idx_hbm.at[pl.ds(row0, CHUNK)], t_idx_v)
            pltpu.sync_copy(s_idx_hbm.at[pl.ds(row0, CHUNK)], s_idx_v)
            t_cp = pltpu.async_copy(t_tab_hbm.at[t_idx_v], t_rows_v, t_sem)
            s_cp = pltpu.async_copy(s_tab_hbm.at[s_idx_v], s_rows_v, s_sem)
            t_cp.wait()
            s_cp.wait()
            pltpu.sync_copy(t_rows_v, out_hbm.at[pl.ds(row0, CHUNK), pl.ds(0, TOPIC_DIM)])
            pltpu.sync_copy(s_rows_v, out_hbm.at[pl.ds(row0, CHUNK), pl.ds(TOPIC_DIM, SUBTOPIC_DIM)])
            return carry

        lax.fori_loop(0, n_chunks, body, 0)

    return enc


def kernel(topic, subtopic, topic_table, subtopic_table):
    enc = _build()
    return enc(topic.astype(jnp.int32), subtopic.astype(jnp.int32),
               topic_table, subtopic_table)


# speedup vs baseline: 1.0095x; 1.0095x over previous
import functools

import jax
import jax.numpy as jnp
from jax import lax
from jax.experimental import pallas as pl
from jax.experimental.pallas import tpu as pltpu
from jax.experimental.pallas import tpu_sc as plsc

BATCH = 16384
TOPIC_DIM = 128
SUBTOPIC_DIM = 64
OUT_DIM = TOPIC_DIM + SUBTOPIC_DIM
CHUNK = 128


@functools.cache
def _build():
    info = plsc.get_sparse_core_info()
    num_workers = info.num_cores * info.num_subcores
    rows_per_worker = BATCH // num_workers
    n_chunks = rows_per_worker // CHUNK
    mesh = plsc.VectorSubcoreMesh(core_axis_name="c", subcore_axis_name="s")

    scratch = [
        pltpu.VMEM((n_chunks, CHUNK), jnp.int32),
        pltpu.VMEM((n_chunks, CHUNK), jnp.int32),
        [pltpu.VMEM((CHUNK, TOPIC_DIM), jnp.float32)] * n_chunks,
        [pltpu.VMEM((CHUNK, SUBTOPIC_DIM), jnp.float32)] * n_chunks,
        [pltpu.SemaphoreType.DMA] * n_chunks,
        [pltpu.SemaphoreType.DMA] * n_chunks,
        pltpu.SemaphoreType.DMA,
    ]

    @functools.partial(
        pl.kernel,
        mesh=mesh,
        out_type=jax.ShapeDtypeStruct((BATCH, OUT_DIM), jnp.float32),
        scratch_types=scratch,
        compiler_params=pltpu.CompilerParams(use_tc_tiling_on_sc=False),
    )
    def enc(t_idx_hbm, s_idx_hbm, t_tab_hbm, s_tab_hbm, out_hbm,
            t_idx_v, s_idx_v, t_rows, s_rows, t_sems, s_sems, w_sem):
        wid = lax.axis_index("s") * info.num_cores + lax.axis_index("c")
        base = wid * rows_per_worker
        idx_row0 = wid * n_chunks

        pltpu.sync_copy(t_idx_hbm.at[pl.ds(idx_row0, n_chunks)], t_idx_v)
        pltpu.sync_copy(s_idx_hbm.at[pl.ds(idx_row0, n_chunks)], s_idx_v)

        t_cps = []
        s_cps = []
        for c in range(n_chunks):
            t_cps.append(pltpu.async_copy(
                t_tab_hbm.at[t_idx_v.at[c]], t_rows[c], t_sems[c]))
            s_cps.append(pltpu.async_copy(
                s_tab_hbm.at[s_idx_v.at[c]], s_rows[c], s_sems[c]))

        w_cps = []
        for c in range(n_chunks):
            row0 = base + c * CHUNK
            t_cps[c].wait()
            w_cps.append(pltpu.async_copy(
                t_rows[c], out_hbm.at[pl.ds(row0, CHUNK), pl.ds(0, TOPIC_DIM)],
                w_sem))
            s_cps[c].wait()
            w_cps.append(pltpu.async_copy(
                s_rows[c],
                out_hbm.at[pl.ds(row0, CHUNK), pl.ds(TOPIC_DIM, SUBTOPIC_DIM)],
                w_sem))

        for cp in w_cps:
            cp.wait()

    return enc


def kernel(topic, subtopic, topic_table, subtopic_table):
    enc = _build()
    n_rows = BATCH // CHUNK
    return enc(topic.astype(jnp.int32).reshape(n_rows, CHUNK),
               subtopic.astype(jnp.int32).reshape(n_rows, CHUNK),
               topic_table, subtopic_table)


# speedup vs baseline: 1.0245x; 1.0149x over previous
import functools

import jax
import jax.numpy as jnp
from jax import lax
from jax.experimental import pallas as pl
from jax.experimental.pallas import tpu as pltpu
from jax.experimental.pallas import tpu_sc as plsc

BATCH = 16384
TOPIC_DIM = 128
SUBTOPIC_DIM = 64
OUT_DIM = TOPIC_DIM + SUBTOPIC_DIM
CHUNK = 128


@functools.cache
def _build():
    info = plsc.get_sparse_core_info()
    num_workers = info.num_cores * info.num_subcores
    rows_per_worker = BATCH // num_workers
    n_chunks = rows_per_worker // CHUNK
    mesh = plsc.VectorSubcoreMesh(core_axis_name="c", subcore_axis_name="s")

    scratch = [
        pltpu.VMEM((n_chunks, CHUNK), jnp.int32),
        pltpu.VMEM((n_chunks, CHUNK), jnp.int32),
        pltpu.VMEM((rows_per_worker, TOPIC_DIM), jnp.float32),
        pltpu.VMEM((rows_per_worker, SUBTOPIC_DIM), jnp.float32),
        pltpu.SemaphoreType.DMA,
        pltpu.SemaphoreType.DMA,
        pltpu.SemaphoreType.DMA,
    ]

    @functools.partial(
        pl.kernel,
        mesh=mesh,
        out_type=jax.ShapeDtypeStruct((BATCH, OUT_DIM), jnp.float32),
        scratch_types=scratch,
        compiler_params=pltpu.CompilerParams(use_tc_tiling_on_sc=False),
    )
    def enc(t_idx_hbm, s_idx_hbm, t_tab_hbm, s_tab_hbm, out_hbm,
            t_idx_v, s_idx_v, t_rows, s_rows, t_sem, s_sem, w_sem):
        wid = lax.axis_index("s") * info.num_cores + lax.axis_index("c")
        base = wid * rows_per_worker
        idx_row0 = wid * n_chunks

        pltpu.sync_copy(t_idx_hbm.at[pl.ds(idx_row0, n_chunks)], t_idx_v)
        pltpu.sync_copy(s_idx_hbm.at[pl.ds(idx_row0, n_chunks)], s_idx_v)

        t_cps = []
        s_cps = []
        for c in range(n_chunks):
            rows = pl.ds(c * CHUNK, CHUNK)
            t_cps.append(pltpu.async_copy(
                t_tab_hbm.at[t_idx_v.at[c]], t_rows.at[rows], t_sem))
            s_cps.append(pltpu.async_copy(
                s_tab_hbm.at[s_idx_v.at[c]], s_rows.at[rows], s_sem))

        for cp in t_cps:
            cp.wait()
        w1 = pltpu.async_copy(
            t_rows, out_hbm.at[pl.ds(base, rows_per_worker), pl.ds(0, TOPIC_DIM)],
            w_sem)
        for cp in s_cps:
            cp.wait()
        w2 = pltpu.async_copy(
            s_rows,
            out_hbm.at[pl.ds(base, rows_per_worker),
                       pl.ds(TOPIC_DIM, SUBTOPIC_DIM)],
            w_sem)
        w1.wait()
        w2.wait()

    return enc


def kernel(topic, subtopic, topic_table, subtopic_table):
    enc = _build()
    n_rows = BATCH // CHUNK
    return enc(topic.astype(jnp.int32).reshape(n_rows, CHUNK),
               subtopic.astype(jnp.int32).reshape(n_rows, CHUNK),
               topic_table, subtopic_table)
